# baseline (speedup 1.0000x reference)
import functools

import jax
import jax.numpy as jnp
from jax import lax
from jax.experimental import pallas as pl
from jax.experimental.pallas import tpu as pltpu
from jax.experimental.pallas import tpu_sc as plsc

D = 64
SCALE = 8.0
TB = 2


@functools.lru_cache(maxsize=None)
def _make_gather(T, NB, V2):
    info = plsc.get_sparse_core_info()
    NC, NS, L = info.num_cores, info.num_subcores, info.num_lanes
    NW = NC * NS
    BW = NB // NW
    NIT = T // TB
    NG = BW // L
    mesh = plsc.VectorSubcoreMesh(core_axis_name="c", subcore_axis_name="s")

    @functools.partial(
        pl.kernel,
        mesh=mesh,
        compiler_params=pltpu.CompilerParams(
            use_tc_tiling_on_sc=True, needs_layout_passes=False
        ),
        out_type=jax.ShapeDtypeStruct((T, D, NB), jnp.float32),
        scratch_types=[
            pltpu.VMEM((2, TB, BW), jnp.int32),
            pltpu.VMEM((2, TB, BW), jnp.int32),
            pltpu.VMEM((2, TB * BW, 2 * D), jnp.float32),
            pltpu.VMEM((2, TB, D, BW), jnp.float32),
            pltpu.SemaphoreType.DMA,
            pltpu.SemaphoreType.DMA,
        ],
    )
    def k(tt_hbm, tp_hbm, out_hbm, idx_v, pidx_v, gbuf, obuf, gsem, osem):
        wid = lax.axis_index("s") * NC + lax.axis_index("c")
        b0 = wid * BW

        def gather_copies(b):
            return [
                pltpu.make_async_copy(
                    tp_hbm.at[pidx_v.at[b].at[t]],
                    gbuf.at[b].at[pl.ds(t * BW, BW)],
                    gsem,
                )
                for t in range(TB)
            ]

        def fire_gather(g, b):
            pltpu.sync_copy(
                tt_hbm.at[pl.ds(g * TB, TB), pl.ds(b0, BW)], idx_v.at[b]
            )
            for t in range(TB):
                for gr in range(NG):
                    sl = pl.ds(gr * L, L)
                    pidx_v[b, t, sl] = lax.shift_right_logical(
                        idx_v[b, t, sl], 1
                    )
            for c in gather_copies(b):
                c.start()

        def wait_gather(b):
            for c in gather_copies(b):
                c.wait()

        fire_gather(0, 0)

        def stage(g, b):
            wait_gather(b)

            @pl.when(g > 0)
            def _():
                pltpu.make_async_copy(
                    obuf.at[1 - b],
                    out_hbm.at[pl.ds((g - 1) * TB, TB), :, pl.ds(b0, BW)],
                    osem,
                ).wait()

            fire_gather(lax.rem(g + 1, NIT), 1 - b)

            lanes = lax.iota(jnp.int32, L)
            for t in range(TB):
                for gr in range(NG):
                    sl = pl.ds(gr * L, L)
                    par = lax.bitwise_and(idx_v[b, t, sl], 1)
                    rows = t * BW + gr * L + lanes
                    colbase = par * D

                    def cbody(c0, carry, sl=sl, rows=rows, colbase=colbase, t=t):
                        for u in range(8):
                            c = c0 * 8 + u
                            vals = plsc.load_gather(
                                gbuf.at[b], [rows, colbase + c]
                            )
                            obuf[b, t, c, sl] = vals * SCALE
                        return carry

                    lax.fori_loop(0, D // 8, cbody, 0)

            pltpu.async_copy(
                obuf.at[b],
                out_hbm.at[pl.ds(g * TB, TB), :, pl.ds(b0, BW)],
                osem,
            )

        def outer(i2, carry):
            for b in range(2):
                stage(i2 * 2 + b, b)
            return carry

        lax.fori_loop(0, NIT // 2, outer, 0)

        wait_gather(0)
        pltpu.make_async_copy(
            obuf.at[1],
            out_hbm.at[pl.ds((NIT - 1) * TB, TB), :, pl.ds(b0, BW)],
            osem,
        ).wait()

    return k


def kernel(token_ids, table):
    NB, T = token_ids.shape
    V, _ = table.shape
    tt = jnp.transpose(token_ids)
    tp = table.reshape(V // 2, 2 * D)
    out_t = _make_gather(T, NB, V // 2)(tt, tp)
    return jnp.transpose(out_t, (2, 0, 1))


# speedup vs baseline: 2.2494x; 2.2494x over previous
import functools

import jax
import jax.numpy as jnp
from jax import lax
from jax.experimental import pallas as pl
from jax.experimental.pallas import tpu as pltpu
from jax.experimental.pallas import tpu_sc as plsc

D = 64
SCALE = 8.0
ST = 256
SUB = 128


@functools.lru_cache(maxsize=None)
def _make_gather(B):
    info = plsc.get_sparse_core_info()
    NC, NS, L = info.num_cores, info.num_subcores, info.num_lanes
    NW = NC * NS
    PW = B // NW
    NSTG = PW // ST
    mesh = plsc.VectorSubcoreMesh(core_axis_name="c", subcore_axis_name="s")

    @functools.partial(
        pl.kernel,
        mesh=mesh,
        compiler_params=pltpu.CompilerParams(
            use_tc_tiling_on_sc=True, needs_layout_passes=False
        ),
        out_type=jax.ShapeDtypeStruct((B, D), jnp.float32),
        scratch_types=[
            pltpu.VMEM((PW,), jnp.int32),
            pltpu.VMEM((2, ST, 2 * D), jnp.float32),
            pltpu.VMEM((ST, D), jnp.float32),
            pltpu.SemaphoreType.DMA,
            pltpu.SemaphoreType.DMA,
        ],
    )
    def k(idx_hbm, tp_hbm, out_hbm, idx_all, gbuf, obuf, gsem, osem):
        wid = lax.axis_index("s") * NC + lax.axis_index("c")
        base = wid * PW

        pltpu.sync_copy(idx_hbm.at[pl.ds(base, PW)], idx_all)

        def gather_copies(g, b):
            return [
                pltpu.make_async_copy(
                    tp_hbm.at[idx_all.at[pl.ds(g * ST + o, SUB)]],
                    gbuf.at[b].at[pl.ds(o, SUB)],
                    gsem,
                )
                for o in range(0, ST, SUB)
            ]

        def fire_gather(g, b):
            for c in gather_copies(g, b):
                c.start()

        def wait_gather(g, b):
            for c in gather_copies(g, b):
                c.wait()

        def out_copy(g):
            return pltpu.make_async_copy(
                obuf, out_hbm.at[pl.ds(base + g * ST, ST)], osem
            )

        fire_gather(0, 0)

        def stage(g, b):
            wait_gather(g, b)
            fire_gather(lax.rem(g + 1, NSTG), 1 - b)

            @pl.when(g > 0)
            def _():
                out_copy(g - 1).wait()

            def token(t4, carry2):
                for u in range(4):
                    t = t4 * 4 + u
                    for c in range(D // L):
                        sl = pl.ds(c * L, L)
                        obuf[t, sl] = gbuf[b, t, sl] * SCALE
                return carry2

            lax.fori_loop(0, ST // 4, token, 0)
            out_copy(g).start()

        def outer(i2, carry):
            for b in range(2):
                stage(i2 * 2 + b, b)
            return carry

        lax.fori_loop(0, NSTG // 2, outer, 0)

        wait_gather(0, 0)
        out_copy(NSTG - 1).wait()

    return k


def kernel(token_ids, table):
    NB, T = token_ids.shape
    tp = jnp.pad(table, ((0, 0), (0, D)))
    out = _make_gather(NB * T)(token_ids.reshape(NB * T), tp)
    return out.reshape(NB, T, D)
